# baseline (speedup 1.0000x reference)
import functools

import jax
import jax.numpy as jnp
from jax import lax
from jax.experimental import pallas as pl
from jax.experimental.pallas import tpu as pltpu
from jax.experimental.pallas import tpu_sc as plsc

_VOCAB = 100000
_EMBED = 64
_B = 4096
_S = 50
_NC = 2
_NS = 16
_L = 16
_G = 8
_CB = 2048
_NFULL = _VOCAB // _CB
_TAILA = 1664
_TAILB = 32

_LOG1P_C = (
    9.999450501e-01, -4.970314631e-01, 3.065628442e-01, -1.578400499e-01,
    4.155156826e-02,
)


def _softplus16(x):
    t = jnp.exp(jnp.minimum(x, -x))
    q = jnp.full((_L,), _LOG1P_C[-1], jnp.float32)
    for c in _LOG1P_C[-2::-1]:
        q = q * t + c
    return jnp.maximum(x, 0.0) + t * q


def _pack16(a, b):
    return plsc.bitcast(
        plsc.pack(a, b, format=plsc.PackFormat.INTERLEAVED), jnp.float32)


def _unpack16(w):
    a, b = plsc.unpack(
        plsc.bitcast(w, jnp.bfloat16), format=plsc.PackFormat.INTERLEAVED)
    return a.astype(jnp.float32), b.astype(jnp.float32)


def _make_tgather():
    mesh = plsc.VectorSubcoreMesh(
        core_axis_name="c", subcore_axis_name="s",
        num_cores=_NC, num_subcores=_NS)

    @functools.partial(
        pl.kernel,
        out_type=jax.ShapeDtypeStruct((_S, _EMBED, _B), jnp.float32),
        mesh=mesh,
        compiler_params=pltpu.CompilerParams(needs_layout_passes=False),
        scratch_types=[
            pltpu.VMEM((_VOCAB,), jnp.float32),
            [pltpu.VMEM((_B,), jnp.int32) for _ in range(2)],
            [pltpu.VMEM((1, 2, _B), jnp.float32) for _ in range(2)],
            [pltpu.VMEM((1, _CB), jnp.float32) for _ in range(2)],
            pltpu.VMEM((1, _TAILA), jnp.float32),
            pltpu.VMEM((_TAILB,), jnp.float32),
            [pltpu.SemaphoreType.DMA for _ in range(2)],
            [pltpu.SemaphoreType.DMA for _ in range(2)],
            [pltpu.SemaphoreType.DMA for _ in range(2)],
        ],
    )
    def tgather(rawT_hbm, idxT_hbm, rawtail_hbm, out_hbm, tbl_v, idx_bufs,
                out_bufs, bbufs, tailbuf, tail2, sem_idx, sem_out, sem_b):
        wid = lax.axis_index("s") * _NC + lax.axis_index("c")
        e0 = wid * 2

        pltpu.async_copy(idxT_hbm.at[0], idx_bufs[0], sem_idx[0])
        pltpu.async_copy(idxT_hbm.at[1], idx_bufs[1], sem_idx[1])
        with jax.named_scope("tbl_load"):
            pltpu.sync_copy(rawT_hbm.at[e0], tbl_v)
        pltpu.async_copy(rawT_hbm.at[pl.ds(e0 + 1, 1), pl.ds(0, _CB)], bbufs[0], sem_b[0])
        pltpu.async_copy(rawT_hbm.at[pl.ds(e0 + 1, 1), pl.ds(_CB, _CB)], bbufs[1], sem_b[1])

        def _pack_chunk(cw, bbuf, nwords):
            @pl.loop(0, nwords // (_L * _G))
            def _pk(g):
                base = g * (_L * _G)
                offs = [base + j * _L for j in range(_G)]
                avs = [tbl_v[pl.ds(cw + o, _L)] for o in offs]
                bvs = [bbuf[0, pl.ds(o, _L)] for o in offs]
                pas = [_softplus16(a) for a in avs]
                pbs = [_softplus16(b) for b in bvs]
                pks = [_pack16(pa, pb) for pa, pb in zip(pas, pbs)]
                for j in range(_G):
                    tbl_v[pl.ds(cw + offs[j], _L)] = pks[j]

        with jax.named_scope("softplus_pack"):
            @pl.loop(0, _NFULL // 2)
            def _pair(p):
                for par in range(2):
                    c = 2 * p + par
                    bbuf, sem = bbufs[par], sem_b[par]
                    pltpu.make_async_copy(
                        rawT_hbm.at[pl.ds(e0 + 1, 1), pl.ds(0, _CB)], bbuf,
                        sem).wait()
                    _pack_chunk(c * _CB, bbuf, _CB)

                    @pl.when(c + 2 < _NFULL)
                    def _(c=c, bbuf=bbuf, sem=sem):
                        pltpu.async_copy(
                            rawT_hbm.at[pl.ds(e0 + 1, 1),
                                        pl.ds((c + 2) * _CB, _CB)],
                            bbuf, sem)

            pltpu.sync_copy(
                rawT_hbm.at[pl.ds(e0 + 1, 1), pl.ds(_NFULL * _CB, _TAILA)],
                tailbuf)
            _pack_chunk(_NFULL * _CB, tailbuf, _TAILA)
            pltpu.sync_copy(rawtail_hbm.at[e0 + 1], tail2)
            t2base = _NFULL * _CB + _TAILA
            for j in range(_TAILB // _L):
                a = tbl_v[pl.ds(t2base + j * _L, _L)]
                b = tail2[pl.ds(j * _L, _L)]
                tbl_v[pl.ds(t2base + j * _L, _L)] = _pack16(
                    _softplus16(a), _softplus16(b))

        def _gather_slab(s, bi):
            idx_v, out2, sem_o = idx_bufs[bi], out_bufs[bi], sem_out[bi]
            pltpu.make_async_copy(idxT_hbm.at[s], idx_v, sem_idx[bi]).wait()

            @pl.when(s >= 2)
            def _():
                pltpu.make_async_copy(
                    out2, out_hbm.at[pl.ds(s, 1), pl.ds(e0, 2)], sem_o).wait()

            @pl.loop(0, _B // (_L * _G))
            def _g(g):
                base = g * (_L * _G)
                ivs = [idx_v[pl.ds(base + j * _L, _L)] for j in range(_G)]
                ws = [plsc.load_gather(tbl_v, [iv]) for iv in ivs]
                abs_ = [_unpack16(w) for w in ws]
                for j in range(_G):
                    out2[0, 0, pl.ds(base + j * _L, _L)] = abs_[j][0]
                    out2[0, 1, pl.ds(base + j * _L, _L)] = abs_[j][1]

            pltpu.async_copy(out2, out_hbm.at[pl.ds(s, 1), pl.ds(e0, 2)], sem_o)

            @pl.when(s + 2 < _S)
            def _():
                pltpu.async_copy(idxT_hbm.at[s + 2], idx_v, sem_idx[bi])

        @pl.loop(0, _S // 2)
        def _slab2(i):
            _gather_slab(2 * i, 0)
            _gather_slab(2 * i + 1, 1)

        pltpu.make_async_copy(
            out_bufs[0], out_hbm.at[pl.ds(_S - 2, 1), pl.ds(e0, 2)], sem_out[0]).wait()
        pltpu.make_async_copy(
            out_bufs[1], out_hbm.at[pl.ds(_S - 1, 1), pl.ds(e0, 2)], sem_out[1]).wait()

    return tgather


def kernel(idx, raw):
    rawT = jnp.transpose(raw)
    idxT = jnp.transpose(idx.astype(jnp.int32))
    raw_tail = jnp.transpose(raw[_NFULL * _CB + _TAILA:, :])
    outT = _make_tgather()(rawT, idxT, raw_tail)
    return jnp.transpose(outT, (2, 0, 1))
